# baseline (speedup 1.0000x reference)
import jax
import jax.numpy as jnp
from jax.experimental import pallas as pl
from jax.experimental.pallas import tpu as pltpu

_DEG = 32
_BLOCK = 320


def _fused_body(self_ref, neigh_ref, wn_ref, ws_ref, out_ref):
    neigh_mean = jnp.sum(neigh_ref[...], axis=1) * (1.0 / _DEG)
    acc = jnp.dot(self_ref[...], ws_ref[...], preferred_element_type=jnp.float32)
    acc = acc + jnp.dot(neigh_mean, wn_ref[...], preferred_element_type=jnp.float32)
    out_ref[...] = jnp.maximum(acc, 0.0)


def kernel(self_vecs, neigh_vecs, neigh_weights, self_weights):
    n, d_in = self_vecs.shape
    deg = neigh_vecs.shape[1]
    d_out = neigh_weights.shape[1]
    return pl.pallas_call(
        _fused_body,
        grid=(pl.cdiv(n, _BLOCK),),
        in_specs=[
            pl.BlockSpec((_BLOCK, d_in), lambda i: (i, 0)),
            pl.BlockSpec((_BLOCK, deg, d_in), lambda i: (i, 0, 0)),
            pl.BlockSpec((d_in, d_out), lambda i: (0, 0)),
            pl.BlockSpec((d_in, d_out), lambda i: (0, 0)),
        ],
        out_specs=pl.BlockSpec((_BLOCK, d_out), lambda i: (i, 0)),
        out_shape=jax.ShapeDtypeStruct((n, d_out), jnp.float32),
        compiler_params=pltpu.CompilerParams(
            dimension_semantics=("arbitrary",),
        ),
    )(self_vecs, neigh_vecs, neigh_weights, self_weights)


# speedup vs baseline: 1.0452x; 1.0452x over previous
import jax
import jax.numpy as jnp
from jax.experimental import pallas as pl
from jax.experimental.pallas import tpu as pltpu

_DEG = 32
_BLOCK = 480


def _fused_body(self_ref, neigh_ref, wn_ref, ws_ref, out_ref):
    neigh_mean = jnp.sum(neigh_ref[...], axis=1) * (1.0 / _DEG)
    acc = jnp.dot(self_ref[...], ws_ref[...], preferred_element_type=jnp.float32)
    acc = acc + jnp.dot(neigh_mean, wn_ref[...], preferred_element_type=jnp.float32)
    out_ref[...] = jnp.maximum(acc, 0.0)


def kernel(self_vecs, neigh_vecs, neigh_weights, self_weights):
    n, d_in = self_vecs.shape
    deg = neigh_vecs.shape[1]
    d_out = neigh_weights.shape[1]
    return pl.pallas_call(
        _fused_body,
        grid=(pl.cdiv(n, _BLOCK),),
        in_specs=[
            pl.BlockSpec((_BLOCK, d_in), lambda i: (i, 0)),
            pl.BlockSpec((_BLOCK, deg, d_in), lambda i: (i, 0, 0)),
            pl.BlockSpec((d_in, d_out), lambda i: (0, 0)),
            pl.BlockSpec((d_in, d_out), lambda i: (0, 0)),
        ],
        out_specs=pl.BlockSpec((_BLOCK, d_out), lambda i: (i, 0)),
        out_shape=jax.ShapeDtypeStruct((n, d_out), jnp.float32),
        compiler_params=pltpu.CompilerParams(
            dimension_semantics=("arbitrary",),
        ),
    )(self_vecs, neigh_vecs, neigh_weights, self_weights)


# speedup vs baseline: 1.0491x; 1.0037x over previous
import jax
import jax.numpy as jnp
from jax.experimental import pallas as pl
from jax.experimental.pallas import tpu as pltpu

_DEG = 32
_BLOCK = 400


def _fused_body(self_ref, neigh_ref, wn_ref, ws_ref, out_ref):
    neigh_mean = jnp.sum(neigh_ref[...], axis=1) * (1.0 / _DEG)
    acc = jnp.dot(self_ref[...], ws_ref[...], preferred_element_type=jnp.float32)
    acc = acc + jnp.dot(neigh_mean, wn_ref[...], preferred_element_type=jnp.float32)
    out_ref[...] = jnp.maximum(acc, 0.0)


def kernel(self_vecs, neigh_vecs, neigh_weights, self_weights):
    n, d_in = self_vecs.shape
    deg = neigh_vecs.shape[1]
    d_out = neigh_weights.shape[1]
    return pl.pallas_call(
        _fused_body,
        grid=(pl.cdiv(n, _BLOCK),),
        in_specs=[
            pl.BlockSpec((_BLOCK, d_in), lambda i: (i, 0)),
            pl.BlockSpec((_BLOCK, deg, d_in), lambda i: (i, 0, 0)),
            pl.BlockSpec((d_in, d_out), lambda i: (0, 0)),
            pl.BlockSpec((d_in, d_out), lambda i: (0, 0)),
        ],
        out_specs=pl.BlockSpec((_BLOCK, d_out), lambda i: (i, 0)),
        out_shape=jax.ShapeDtypeStruct((n, d_out), jnp.float32),
        compiler_params=pltpu.CompilerParams(
            dimension_semantics=("parallel",),
        ),
    )(self_vecs, neigh_vecs, neigh_weights, self_weights)
